# baseline (speedup 1.0000x reference)
import functools

import jax
import jax.numpy as jnp
from jax import lax
from jax.experimental import pallas as pl
from jax.experimental.pallas import tpu as pltpu
from jax.experimental.pallas import tpu_sc as plsc

N = 10000
E = 320000
D = 128
G = 64

NC = 2
NS = 16
NW = NC * NS

KW = 128
STEPS = 80
E_PAD = NW * STEPS * KW
N_PAD = 10240
RB = N_PAD // 128
RPS = N_PAD // NS

_f32 = jnp.float32


def _sc_mesh():
  return plsc.VectorSubcoreMesh(core_axis_name="c", subcore_axis_name="s",
                                num_cores=NC, num_subcores=NS)


def _sc_degree_body(dst_hbm, deg_out, idx_v, ones_v, zero_v, deg_sh, sem):
  c = lax.axis_index("c")
  s = lax.axis_index("s")
  wid = s * NC + c

  for j in range(8):
    ones_v[pl.ds(16 * j, 16)] = jnp.ones((16,), _f32)
  for j in range(RPS // 16):
    zero_v[pl.ds(16 * j, 16)] = jnp.zeros((16,), _f32)
  pltpu.sync_copy(zero_v, deg_sh.at[pl.ds(s * RPS, RPS)])
  plsc.subcore_barrier()

  pltpu.sync_copy(dst_hbm.at[wid], idx_v)
  for g in range(STEPS // 8):
    descs = []
    for j in range(8):
      descs.append(
          pltpu.async_copy(ones_v, deg_sh.at[idx_v.at[8 * g + j]], sem,
                           add=True))
    for d in descs:
      d.wait()
  plsc.subcore_barrier()

  pltpu.sync_copy(deg_sh.at[pl.ds(s * RPS, RPS)],
                  deg_out.at[c, pl.ds(s * RPS, RPS)])


def _sc_degree(dst3):
  return pl.kernel(
      _sc_degree_body,
      out_type=jax.ShapeDtypeStruct((NC, N_PAD), _f32),
      mesh=_sc_mesh(),
      scratch_types=[
          pltpu.VMEM((STEPS, KW), jnp.int32),
          pltpu.VMEM((KW,), _f32),
          pltpu.VMEM((RPS,), _f32),
          pltpu.VMEM_SHARED((N_PAD,), _f32),
          pltpu.SemaphoreType.DMA,
      ],
  )(dst3)


PHASE = 40


def _sc_aggregate_body(zp_hbm, src_hbm, dst_hbm, acc_out,
                       src_idx, dst_idx, rows, acc_sh, gsems, ssems):
  c = lax.axis_index("c")
  s = lax.axis_index("s")
  wid = s * NC + c

  def _zero_row(r, carry):
    for j in range(8):
      rows[0, r, pl.ds(16 * j, 16)] = jnp.zeros((16,), _f32)
    return carry
  lax.fori_loop(0, 128, _zero_row, 0)
  for k in range(RPS // 128):
    pltpu.sync_copy(rows.at[0], acc_sh.at[pl.ds(s * RPS + 128 * k, 128)])
  plsc.subcore_barrier()

  gd = [None, None]
  sd = [None, None]
  for phase in range(STEPS // PHASE):
    pltpu.sync_copy(src_hbm.at[wid, pl.ds(PHASE * phase, PHASE)], src_idx)
    pltpu.sync_copy(dst_hbm.at[wid, pl.ds(PHASE * phase, PHASE)], dst_idx)
    for b in range(2):
      gd[b] = pltpu.async_copy(zp_hbm.at[src_idx.at[b]], rows.at[b],
                               gsems[b])
    for it in range(PHASE + 1):
      if it < PHASE:
        b = it % 2
        gd[b].wait()
        sd[b] = pltpu.async_copy(rows.at[b], acc_sh.at[dst_idx.at[it]],
                                 ssems[b], add=True)
      k = it + 1
      if 2 <= k < PHASE:
        bb = k % 2
        sd[bb].wait()
        gd[bb] = pltpu.async_copy(zp_hbm.at[src_idx.at[k]], rows.at[bb],
                                  gsems[bb])
    sd[(PHASE - 2) % 2].wait()
    sd[(PHASE - 1) % 2].wait()
  plsc.subcore_barrier()

  for k in range(RPS // 128):
    pltpu.sync_copy(acc_sh.at[pl.ds(s * RPS + 128 * k, 128)],
                    acc_out.at[c, pl.ds(s * RPS + 128 * k, 128)])


def _sc_aggregate(zp, src3, dst3):
  body = lambda zp_hbm, src_hbm, dst_hbm, acc_out, src_idx, dst_idx, rows, \
      acc_sh, g0, g1, s0, s1: _sc_aggregate_body(
          zp_hbm, src_hbm, dst_hbm, acc_out, src_idx, dst_idx, rows,
          acc_sh, [g0, g1], [s0, s1])
  return pl.kernel(
      body,
      out_type=jax.ShapeDtypeStruct((NC, N_PAD, D), _f32),
      mesh=_sc_mesh(),
      scratch_types=[
          pltpu.VMEM((PHASE, KW), jnp.int32),
          pltpu.VMEM((PHASE, KW), jnp.int32),
          pltpu.VMEM((2, KW, D), _f32),
          pltpu.VMEM_SHARED((N_PAD, D), _f32),
      ] + [pltpu.SemaphoreType.DMA] * 4,
  )(zp, src3, dst3)


_DOT = functools.partial(jnp.dot, precision=lax.Precision.HIGHEST,
                         preferred_element_type=_f32)


def _row_iota():
  return lax.broadcasted_iota(jnp.int32, (128, 128), 0)


def _row_mask(i):
  return (i * 128 + _row_iota()) < N


def _tc_embed_body(x_ref, deg_ref, wemb_ref, bemb_ref, w0_ref,
                   zp_ref, dinv_ref):
  i = pl.program_id(0)
  degsum = (deg_ref[0, pl.ds(i, 1), :] + deg_ref[1, pl.ds(i, 1), :] + 1.0)
  dinv = lax.rsqrt(degsum)
  row = _row_iota()
  col = lax.broadcasted_iota(jnp.int32, (128, 128), 1)
  diag = jnp.where(row == col, jnp.broadcast_to(dinv, (128, 128)), 0.0)
  dinv_col = _DOT(diag, jnp.ones((128, 8), _f32))
  dinv_ref[...] = dinv_col
  h = _DOT(x_ref[...], wemb_ref[...]) + bemb_ref[...]
  z = _DOT(h, w0_ref[...])
  zp_ref[...] = jnp.where(_row_mask(i), dinv_col[:, :1] * z, 0.0)


def _tc_embed(x_pad, deg2, w_emb, b_emb, w0):
  return pl.pallas_call(
      _tc_embed_body,
      grid=(RB,),
      in_specs=[
          pl.BlockSpec((128, D), lambda i: (i, 0)),
          pl.BlockSpec((NC, RB, 128), lambda i: (0, 0, 0)),
          pl.BlockSpec((D, D), lambda i: (0, 0)),
          pl.BlockSpec((1, D), lambda i: (0, 0)),
          pl.BlockSpec((D, D), lambda i: (0, 0)),
      ],
      out_specs=[
          pl.BlockSpec((128, D), lambda i: (i, 0)),
          pl.BlockSpec((128, 8), lambda i: (i, 0)),
      ],
      out_shape=[
          jax.ShapeDtypeStruct((N_PAD, D), _f32),
          jax.ShapeDtypeStruct((N_PAD, 8), _f32),
      ],
      compiler_params=pltpu.CompilerParams(
          dimension_semantics=("arbitrary",)),
  )(x_pad, deg2, w_emb, b_emb, w0)


def _tc_layer_body(acc_ref, zp_ref, dinv_ref, b_ref, g_ref, be_ref, w_ref,
                   out_ref, stats_ref):
  p = pl.program_id(0)
  i = pl.program_id(1)
  dinv_col = dinv_ref[...][:, :1]
  pre = dinv_col * (acc_ref[0] + acc_ref[1] + zp_ref[...]) + b_ref[...]

  @pl.when(p == 0)
  def _():
    @pl.when(i == 0)
    def _():
      stats_ref[...] = jnp.zeros((8, 128), _f32)
    pm = jnp.where(_row_mask(i), pre, 0.0)
    stats_ref[pl.ds(0, 1), :] += jnp.sum(pm, axis=0, keepdims=True)
    stats_ref[pl.ds(1, 1), :] += jnp.sum(pm * pm, axis=0, keepdims=True)

  @pl.when(p == 1)
  def _():
    inv_n = 1.0 / N
    mu = stats_ref[pl.ds(0, 1), :] * inv_n
    var = stats_ref[pl.ds(1, 1), :] * inv_n - mu * mu
    scale = g_ref[...] * lax.rsqrt(var + 1e-5)
    h = jnp.maximum((pre - mu) * scale + be_ref[...], 0.0)
    z = _DOT(h, w_ref[...])
    out_ref[...] = jnp.where(_row_mask(i), dinv_col * z, 0.0)


def _tc_layer(acc, zp, dinv, b, g, be, w):
  return pl.pallas_call(
      _tc_layer_body,
      grid=(2, RB),
      in_specs=[
          pl.BlockSpec((NC, 128, D), lambda p, i: (0, i, 0)),
          pl.BlockSpec((128, D), lambda p, i: (i, 0)),
          pl.BlockSpec((128, 8), lambda p, i: (i, 0)),
          pl.BlockSpec((1, D), lambda p, i: (0, 0)),
          pl.BlockSpec((1, D), lambda p, i: (0, 0)),
          pl.BlockSpec((1, D), lambda p, i: (0, 0)),
          pl.BlockSpec((D, D), lambda p, i: (0, 0)),
      ],
      out_specs=pl.BlockSpec((128, D), lambda p, i: (i, 0)),
      out_shape=jax.ShapeDtypeStruct((N_PAD, D), _f32),
      scratch_shapes=[pltpu.VMEM((8, 128), _f32)],
      compiler_params=pltpu.CompilerParams(
          dimension_semantics=("arbitrary", "arbitrary")),
  )(acc, zp, dinv, b, g, be, w)


def _tc_pool_body(acc_ref, zp_ref, dinv_ref, b_ref, batch_ref, out_ref,
                  cnt_ref):
  i = pl.program_id(0)
  dinv_col = dinv_ref[...][:, :1]
  ssum = acc_ref[0] + acc_ref[1] + zp_ref[...]
  pre = dinv_col * ssum + b_ref[...]

  brow = batch_ref[pl.ds(i, 1), :]
  gid = lax.broadcasted_iota(jnp.int32, (G, 128), 0)
  onehot = (gid == jnp.broadcast_to(brow, (G, 128))).astype(_f32)

  @pl.when(i == 0)
  def _():
    out_ref[...] = jnp.zeros((G, D), _f32)
    cnt_ref[...] = jnp.zeros((G, 128), _f32)

  out_ref[...] += _DOT(onehot, pre)
  cnt_ref[...] += jnp.broadcast_to(
      jnp.sum(onehot, axis=1, keepdims=True), (G, 128))

  @pl.when(i == RB - 1)
  def _():
    out_ref[...] = out_ref[...] / jnp.maximum(cnt_ref[...], 1.0)


def _tc_pool(acc, zp, dinv, b, batch2):
  return pl.pallas_call(
      _tc_pool_body,
      grid=(RB,),
      in_specs=[
          pl.BlockSpec((NC, 128, D), lambda i: (0, i, 0)),
          pl.BlockSpec((128, D), lambda i: (i, 0)),
          pl.BlockSpec((128, 8), lambda i: (i, 0)),
          pl.BlockSpec((1, D), lambda i: (0, 0)),
          pl.BlockSpec((RB, 128), lambda i: (0, 0)),
      ],
      out_specs=pl.BlockSpec((G, D), lambda i: (0, 0)),
      out_shape=jax.ShapeDtypeStruct((G, D), _f32),
      scratch_shapes=[pltpu.VMEM((G, 128), _f32)],
      compiler_params=pltpu.CompilerParams(
          dimension_semantics=("arbitrary",)),
  )(acc, zp, dinv, b, batch2)


def kernel(x, edge_index, batch, W_emb, b_emb, W0, b0, g0, be0,
           W1, b1, g1, be1, W2, b2):
  n_extra = E_PAD - E
  pad_src = jnp.arange(n_extra, dtype=jnp.int32) % N
  pad_dst = N + 1 + jnp.arange(n_extra, dtype=jnp.int32) % (N_PAD - N - 1)
  src3 = jnp.concatenate([edge_index[0], pad_src]).reshape(NW, STEPS, KW)
  dst3 = jnp.concatenate([edge_index[1], pad_dst]).reshape(NW, STEPS, KW)

  x_pad = jnp.concatenate([x.astype(_f32),
                           jnp.zeros((N_PAD - N, D), _f32)], axis=0)
  batch2 = jnp.concatenate(
      [batch.astype(jnp.int32),
       jnp.full((N_PAD - N,), 127, jnp.int32)]).reshape(RB, 128)

  r1 = lambda v: v.astype(_f32).reshape(1, D)

  deg2 = _sc_degree(dst3).reshape(NC, RB, 128)
  zp0, dinv = _tc_embed(x_pad, deg2, W_emb.astype(_f32), r1(b_emb),
                        W0.astype(_f32))

  acc = _sc_aggregate(zp0, src3, dst3)
  zp1 = _tc_layer(acc, zp0, dinv, r1(b0), r1(g0), r1(be0), W1.astype(_f32))

  acc = _sc_aggregate(zp1, src3, dst3)
  zp2 = _tc_layer(acc, zp1, dinv, r1(b1), r1(g1), r1(be1), W2.astype(_f32))

  acc = _sc_aggregate(zp2, src3, dst3)
  return _tc_pool(acc, zp2, dinv, r1(b2), batch2)


# speedup vs baseline: 1.0425x; 1.0425x over previous
import functools

import jax
import jax.numpy as jnp
from jax import lax
from jax.experimental import pallas as pl
from jax.experimental.pallas import tpu as pltpu
from jax.experimental.pallas import tpu_sc as plsc

N = 10000
E = 320000
D = 128
G = 64

NC = 2
NS = 16
NW = NC * NS

KW = 128
STEPS = 80
E_PAD = NW * STEPS * KW
N_PAD = 10240
RB = N_PAD // 128
RPS = N_PAD // NS

_f32 = jnp.float32


def _sc_mesh():
  return plsc.VectorSubcoreMesh(core_axis_name="c", subcore_axis_name="s",
                                num_cores=NC, num_subcores=NS)


def _sc_degree_body(dst_hbm, deg_out, idx_v, ones_v, zero_v, deg_sh, sem):
  c = lax.axis_index("c")
  s = lax.axis_index("s")
  wid = s * NC + c

  for j in range(8):
    ones_v[pl.ds(16 * j, 16)] = jnp.ones((16,), _f32)
  for j in range(RPS // 16):
    zero_v[pl.ds(16 * j, 16)] = jnp.zeros((16,), _f32)
  pltpu.sync_copy(zero_v, deg_sh.at[pl.ds(s * RPS, RPS)])
  plsc.subcore_barrier()

  pltpu.sync_copy(dst_hbm.at[wid], idx_v)
  for g in range(STEPS // 8):
    descs = []
    for j in range(8):
      descs.append(
          pltpu.async_copy(ones_v, deg_sh.at[idx_v.at[8 * g + j]], sem,
                           add=True))
    for d in descs:
      d.wait()
  plsc.subcore_barrier()

  pltpu.sync_copy(deg_sh.at[pl.ds(s * RPS, RPS)],
                  deg_out.at[c, pl.ds(s * RPS, RPS)])


def _sc_degree(dst3):
  return pl.kernel(
      _sc_degree_body,
      out_type=jax.ShapeDtypeStruct((NC, N_PAD), _f32),
      mesh=_sc_mesh(),
      scratch_types=[
          pltpu.VMEM((STEPS, KW), jnp.int32),
          pltpu.VMEM((KW,), _f32),
          pltpu.VMEM((RPS,), _f32),
          pltpu.VMEM_SHARED((N_PAD,), _f32),
          pltpu.SemaphoreType.DMA,
      ],
  )(dst3)


PHASE = 40


def _sc_aggregate_body(zp_hbm, src_hbm, dst_hbm, acc_out,
                       src_idx, dst_idx, rows, acc_sh, gsems, ssems):
  c = lax.axis_index("c")
  s = lax.axis_index("s")
  wid = s * NC + c

  def _zero_row(r, carry):
    for j in range(8):
      rows[0, r, pl.ds(16 * j, 16)] = jnp.zeros((16,), _f32)
    return carry
  lax.fori_loop(0, 128, _zero_row, 0)
  for k in range(RPS // 128):
    pltpu.sync_copy(rows.at[0], acc_sh.at[pl.ds(s * RPS + 128 * k, 128)])
  plsc.subcore_barrier()

  gd = [None, None]
  sd = [None, None]
  for phase in range(STEPS // PHASE):
    pltpu.sync_copy(src_hbm.at[wid, pl.ds(PHASE * phase, PHASE)], src_idx)
    pltpu.sync_copy(dst_hbm.at[wid, pl.ds(PHASE * phase, PHASE)], dst_idx)
    for b in range(2):
      gd[b] = pltpu.async_copy(zp_hbm.at[src_idx.at[b]], rows.at[b],
                               gsems[b])
    for it in range(PHASE + 1):
      if it < PHASE:
        b = it % 2
        gd[b].wait()
        sd[b] = pltpu.async_copy(rows.at[b], acc_sh.at[dst_idx.at[it]],
                                 ssems[b], add=True)
      k = it + 1
      if 2 <= k < PHASE:
        bb = k % 2
        sd[bb].wait()
        gd[bb] = pltpu.async_copy(zp_hbm.at[src_idx.at[k]], rows.at[bb],
                                  gsems[bb])
    sd[(PHASE - 2) % 2].wait()
    sd[(PHASE - 1) % 2].wait()
  plsc.subcore_barrier()

  for k in range(RPS // 128):
    pltpu.sync_copy(acc_sh.at[pl.ds(s * RPS + 128 * k, 128)],
                    acc_out.at[c, pl.ds(s * RPS + 128 * k, 128)])


def _sc_aggregate(zp, src3, dst3):
  body = lambda zp_hbm, src_hbm, dst_hbm, acc_out, src_idx, dst_idx, rows, \
      acc_sh, g0, g1, s0, s1: _sc_aggregate_body(
          zp_hbm, src_hbm, dst_hbm, acc_out, src_idx, dst_idx, rows,
          acc_sh, [g0, g1], [s0, s1])
  return pl.kernel(
      body,
      out_type=jax.ShapeDtypeStruct((NC, N_PAD, D), _f32),
      mesh=_sc_mesh(),
      scratch_types=[
          pltpu.VMEM((PHASE, KW), jnp.int32),
          pltpu.VMEM((PHASE, KW), jnp.int32),
          pltpu.VMEM((2, KW, D), _f32),
          pltpu.VMEM_SHARED((N_PAD, D), _f32),
      ] + [pltpu.SemaphoreType.DMA] * 4,
  )(zp, src3, dst3)


_DOT = functools.partial(jnp.dot, preferred_element_type=_f32)


def _row_iota():
  return lax.broadcasted_iota(jnp.int32, (128, 128), 0)


def _row_mask(i):
  return (i * 128 + _row_iota()) < N


def _tc_embed_body(x_ref, deg_ref, wemb_ref, bemb_ref, w0_ref,
                   zp_ref, dinv_ref):
  i = pl.program_id(0)
  degsum = (deg_ref[0, pl.ds(i, 1), :] + deg_ref[1, pl.ds(i, 1), :] + 1.0)
  dinv = lax.rsqrt(degsum)
  row = _row_iota()
  col = lax.broadcasted_iota(jnp.int32, (128, 128), 1)
  diag = jnp.where(row == col, jnp.broadcast_to(dinv, (128, 128)), 0.0)
  dinv_col = _DOT(diag, jnp.ones((128, 8), _f32))
  dinv_ref[...] = dinv_col
  h = _DOT(x_ref[...], wemb_ref[...]) + bemb_ref[...]
  z = _DOT(h, w0_ref[...])
  zp_ref[...] = jnp.where(_row_mask(i), dinv_col[:, :1] * z, 0.0)


def _tc_embed(x_pad, deg2, w_emb, b_emb, w0):
  return pl.pallas_call(
      _tc_embed_body,
      grid=(RB,),
      in_specs=[
          pl.BlockSpec((128, D), lambda i: (i, 0)),
          pl.BlockSpec((NC, RB, 128), lambda i: (0, 0, 0)),
          pl.BlockSpec((D, D), lambda i: (0, 0)),
          pl.BlockSpec((1, D), lambda i: (0, 0)),
          pl.BlockSpec((D, D), lambda i: (0, 0)),
      ],
      out_specs=[
          pl.BlockSpec((128, D), lambda i: (i, 0)),
          pl.BlockSpec((128, 8), lambda i: (i, 0)),
      ],
      out_shape=[
          jax.ShapeDtypeStruct((N_PAD, D), _f32),
          jax.ShapeDtypeStruct((N_PAD, 8), _f32),
      ],
      compiler_params=pltpu.CompilerParams(
          dimension_semantics=("arbitrary",)),
  )(x_pad, deg2, w_emb, b_emb, w0)


def _tc_layer_body(acc_ref, zp_ref, dinv_ref, b_ref, g_ref, be_ref, w_ref,
                   out_ref, stats_ref, pre_ref):
  p = pl.program_id(0)
  i = pl.program_id(1)
  dinv_col = dinv_ref[...][:, :1]

  @pl.when(p == 0)
  def _():
    pre = dinv_col * (acc_ref[0] + acc_ref[1] + zp_ref[...]) + b_ref[...]
    pre_ref[pl.ds(i, 1)] = pre.reshape(1, 128, D)

    @pl.when(i == 0)
    def _():
      stats_ref[...] = jnp.zeros((8, 128), _f32)
    pm = jnp.where(_row_mask(i), pre, 0.0)
    stats_ref[pl.ds(0, 1), :] += jnp.sum(pm, axis=0, keepdims=True)
    stats_ref[pl.ds(1, 1), :] += jnp.sum(pm * pm, axis=0, keepdims=True)

  @pl.when(p == 1)
  def _():
    inv_n = 1.0 / N
    mu = stats_ref[pl.ds(0, 1), :] * inv_n
    var = stats_ref[pl.ds(1, 1), :] * inv_n - mu * mu
    scale = g_ref[...] * lax.rsqrt(var + 1e-5)
    pre = pre_ref[pl.ds(i, 1)].reshape(128, D)
    h = jnp.maximum((pre - mu) * scale + be_ref[...], 0.0)
    z = _DOT(h, w_ref[...])
    out_ref[...] = jnp.where(_row_mask(i), dinv_col * z, 0.0)


def _tc_layer(acc, zp, dinv, b, g, be, w):
  return pl.pallas_call(
      _tc_layer_body,
      grid=(2, RB),
      in_specs=[
          pl.BlockSpec((NC, 128, D), lambda p, i: (0, i * (1 - p), 0)),
          pl.BlockSpec((128, D), lambda p, i: (i * (1 - p), 0)),
          pl.BlockSpec((128, 8), lambda p, i: (i, 0)),
          pl.BlockSpec((1, D), lambda p, i: (0, 0)),
          pl.BlockSpec((1, D), lambda p, i: (0, 0)),
          pl.BlockSpec((1, D), lambda p, i: (0, 0)),
          pl.BlockSpec((D, D), lambda p, i: (0, 0)),
      ],
      out_specs=pl.BlockSpec((128, D), lambda p, i: (i, 0)),
      out_shape=jax.ShapeDtypeStruct((N_PAD, D), _f32),
      scratch_shapes=[pltpu.VMEM((8, 128), _f32),
                      pltpu.VMEM((RB, 128, D), _f32)],
      compiler_params=pltpu.CompilerParams(
          dimension_semantics=("arbitrary", "arbitrary")),
  )(acc, zp, dinv, b, g, be, w)


def _tc_pool_body(acc_ref, zp_ref, dinv_ref, b_ref, batch_ref, out_ref,
                  cnt_ref):
  i = pl.program_id(0)
  dinv_col = dinv_ref[...][:, :1]
  ssum = acc_ref[0] + acc_ref[1] + zp_ref[...]
  pre = dinv_col * ssum + b_ref[...]

  brow = batch_ref[pl.ds(i, 1), :]
  gid = lax.broadcasted_iota(jnp.int32, (G, 128), 0)
  onehot = (gid == jnp.broadcast_to(brow, (G, 128))).astype(_f32)

  @pl.when(i == 0)
  def _():
    out_ref[...] = jnp.zeros((G, D), _f32)
    cnt_ref[...] = jnp.zeros((G, 128), _f32)

  out_ref[...] += _DOT(onehot, pre)
  cnt_ref[...] += jnp.broadcast_to(
      jnp.sum(onehot, axis=1, keepdims=True), (G, 128))

  @pl.when(i == RB - 1)
  def _():
    out_ref[...] = out_ref[...] / jnp.maximum(cnt_ref[...], 1.0)


def _tc_pool(acc, zp, dinv, b, batch2):
  return pl.pallas_call(
      _tc_pool_body,
      grid=(RB,),
      in_specs=[
          pl.BlockSpec((NC, 128, D), lambda i: (0, i, 0)),
          pl.BlockSpec((128, D), lambda i: (i, 0)),
          pl.BlockSpec((128, 8), lambda i: (i, 0)),
          pl.BlockSpec((1, D), lambda i: (0, 0)),
          pl.BlockSpec((RB, 128), lambda i: (0, 0)),
      ],
      out_specs=pl.BlockSpec((G, D), lambda i: (0, 0)),
      out_shape=jax.ShapeDtypeStruct((G, D), _f32),
      scratch_shapes=[pltpu.VMEM((G, 128), _f32)],
      compiler_params=pltpu.CompilerParams(
          dimension_semantics=("arbitrary",)),
  )(acc, zp, dinv, b, batch2)


def kernel(x, edge_index, batch, W_emb, b_emb, W0, b0, g0, be0,
           W1, b1, g1, be1, W2, b2):
  n_extra = E_PAD - E
  pad_src = jnp.arange(n_extra, dtype=jnp.int32) % N
  pad_dst = N + 1 + jnp.arange(n_extra, dtype=jnp.int32) % (N_PAD - N - 1)
  src3 = jnp.concatenate([edge_index[0], pad_src]).reshape(NW, STEPS, KW)
  dst3 = jnp.concatenate([edge_index[1], pad_dst]).reshape(NW, STEPS, KW)

  x_pad = jnp.concatenate([x.astype(_f32),
                           jnp.zeros((N_PAD - N, D), _f32)], axis=0)
  batch2 = jnp.concatenate(
      [batch.astype(jnp.int32),
       jnp.full((N_PAD - N,), 127, jnp.int32)]).reshape(RB, 128)

  r1 = lambda v: v.astype(_f32).reshape(1, D)

  deg2 = _sc_degree(dst3).reshape(NC, RB, 128)
  zp0, dinv = _tc_embed(x_pad, deg2, W_emb.astype(_f32), r1(b_emb),
                        W0.astype(_f32))

  acc = _sc_aggregate(zp0, src3, dst3)
  zp1 = _tc_layer(acc, zp0, dinv, r1(b0), r1(g0), r1(be0), W1.astype(_f32))

  acc = _sc_aggregate(zp1, src3, dst3)
  zp2 = _tc_layer(acc, zp1, dinv, r1(b1), r1(g1), r1(be1), W2.astype(_f32))

  acc = _sc_aggregate(zp2, src3, dst3)
  return _tc_pool(acc, zp2, dinv, r1(b2), batch2)


# speedup vs baseline: 1.4428x; 1.3840x over previous
import functools

import jax
import jax.numpy as jnp
from jax import lax
from jax.experimental import pallas as pl
from jax.experimental.pallas import tpu as pltpu
from jax.experimental.pallas import tpu_sc as plsc

N = 10000
E = 320000
D = 128
G = 64

NC = 2
NS = 16
NW = NC * NS

KW = 128
STEPS = 80
E_PAD = NW * STEPS * KW
N_PAD = 10240
RB = N_PAD // 128
RPS = N_PAD // NS

_f32 = jnp.float32


def _sc_mesh():
  return plsc.VectorSubcoreMesh(core_axis_name="c", subcore_axis_name="s",
                                num_cores=NC, num_subcores=NS)


def _sc_degree_body(dst_hbm, deg_out, idx_v, ones_v, zero_v, deg_sh, sem):
  c = lax.axis_index("c")
  s = lax.axis_index("s")
  wid = s * NC + c

  for j in range(8):
    ones_v[pl.ds(16 * j, 16)] = jnp.ones((16,), _f32)
  for j in range(RPS // 16):
    zero_v[pl.ds(16 * j, 16)] = jnp.zeros((16,), _f32)
  pltpu.sync_copy(zero_v, deg_sh.at[pl.ds(s * RPS, RPS)])
  plsc.subcore_barrier()

  pltpu.sync_copy(dst_hbm.at[wid], idx_v)
  for g in range(STEPS // 8):
    descs = []
    for j in range(8):
      descs.append(
          pltpu.async_copy(ones_v, deg_sh.at[idx_v.at[8 * g + j]], sem,
                           add=True))
    for d in descs:
      d.wait()
  plsc.subcore_barrier()

  pltpu.sync_copy(deg_sh.at[pl.ds(s * RPS, RPS)],
                  deg_out.at[c, pl.ds(s * RPS, RPS)])


def _sc_degree(dst3):
  return pl.kernel(
      _sc_degree_body,
      out_type=jax.ShapeDtypeStruct((NC, N_PAD), _f32),
      mesh=_sc_mesh(),
      scratch_types=[
          pltpu.VMEM((STEPS, KW), jnp.int32),
          pltpu.VMEM((KW,), _f32),
          pltpu.VMEM((RPS,), _f32),
          pltpu.VMEM_SHARED((N_PAD,), _f32),
          pltpu.SemaphoreType.DMA,
      ],
  )(dst3)


PHASE = 40


def _sc_aggregate_body(zp_hbm, src_hbm, dst_hbm, acc_out,
                       src_idx, dst_idx, rows, acc_sh, gsems, ssems):
  c = lax.axis_index("c")
  s = lax.axis_index("s")
  wid = s * NC + c

  def _zero_row(r, carry):
    for j in range(8):
      rows[0, r, pl.ds(16 * j, 16)] = jnp.zeros((16,), _f32)
    return carry
  lax.fori_loop(0, 128, _zero_row, 0)
  for k in range(RPS // 128):
    pltpu.sync_copy(rows.at[0], acc_sh.at[pl.ds(s * RPS + 128 * k, 128)])
  plsc.subcore_barrier()

  gd = [None, None]
  sd = [None, None]
  for phase in range(STEPS // PHASE):
    pltpu.sync_copy(src_hbm.at[wid, pl.ds(PHASE * phase, PHASE)], src_idx)
    pltpu.sync_copy(dst_hbm.at[wid, pl.ds(PHASE * phase, PHASE)], dst_idx)
    for b in range(2):
      gd[b] = pltpu.async_copy(zp_hbm.at[src_idx.at[b]], rows.at[b],
                               gsems[b])
    for it in range(PHASE + 1):
      if it < PHASE:
        b = it % 2
        gd[b].wait()
        sd[b] = pltpu.async_copy(rows.at[b], acc_sh.at[dst_idx.at[it]],
                                 ssems[b], add=True)
      k = it + 1
      if 2 <= k < PHASE:
        bb = k % 2
        sd[bb].wait()
        gd[bb] = pltpu.async_copy(zp_hbm.at[src_idx.at[k]], rows.at[bb],
                                  gsems[bb])
    sd[(PHASE - 2) % 2].wait()
    sd[(PHASE - 1) % 2].wait()
  plsc.subcore_barrier()

  for k in range(RPS // 128):
    pltpu.sync_copy(acc_sh.at[pl.ds(s * RPS + 128 * k, 128)],
                    acc_out.at[c, pl.ds(s * RPS + 128 * k, 128)])


def _sc_aggregate(zp, src3, dst3):
  body = lambda zp_hbm, src_hbm, dst_hbm, acc_out, src_idx, dst_idx, rows, \
      acc_sh, g0, g1, s0, s1: _sc_aggregate_body(
          zp_hbm, src_hbm, dst_hbm, acc_out, src_idx, dst_idx, rows,
          acc_sh, [g0, g1], [s0, s1])
  return pl.kernel(
      body,
      out_type=jax.ShapeDtypeStruct((NC, N_PAD, D), _f32),
      mesh=_sc_mesh(),
      scratch_types=[
          pltpu.VMEM((PHASE, KW), jnp.int32),
          pltpu.VMEM((PHASE, KW), jnp.int32),
          pltpu.VMEM((2, KW, D), _f32),
          pltpu.VMEM_SHARED((N_PAD, D), _f32),
      ] + [pltpu.SemaphoreType.DMA] * 4,
  )(zp, src3, dst3)


_DOT = functools.partial(jnp.dot, preferred_element_type=_f32)

BR = 640
GB = N_PAD // BR
SUB = BR // 128


def _row_mask(i):
  return (i * BR + lax.broadcasted_iota(jnp.int32, (BR, 128), 0)) < N


def _tc_embed_body(x_ref, deg_ref, wemb_ref, bemb_ref, w0_ref,
                   zp_ref, dinv_ref):
  i = pl.program_id(0)
  degsum = (deg_ref[0, pl.ds(SUB * i, SUB), :]
            + deg_ref[1, pl.ds(SUB * i, SUB), :] + 1.0)
  dinv = lax.rsqrt(degsum)
  row = lax.broadcasted_iota(jnp.int32, (128, 128), 0)
  col = lax.broadcasted_iota(jnp.int32, (128, 128), 1)
  cols = []
  for k in range(SUB):
    dk = dinv[pl.ds(k, 1) if False else k]
    diag = jnp.where(row == col,
                     jnp.broadcast_to(dk.reshape(1, 128), (128, 128)), 0.0)
    cols.append(_DOT(diag, jnp.ones((128, 8), _f32)))
  dinv_col = jnp.concatenate(cols, axis=0)
  dinv_ref[...] = dinv_col
  h = _DOT(x_ref[...], wemb_ref[...]) + bemb_ref[...]
  z = _DOT(h, w0_ref[...])
  zp_ref[...] = jnp.where(_row_mask(i), dinv_col[:, :1] * z, 0.0)


def _tc_embed(x_pad, deg2, w_emb, b_emb, w0):
  return pl.pallas_call(
      _tc_embed_body,
      grid=(GB,),
      in_specs=[
          pl.BlockSpec((BR, D), lambda i: (i, 0)),
          pl.BlockSpec((NC, RB, 128), lambda i: (0, 0, 0)),
          pl.BlockSpec((D, D), lambda i: (0, 0)),
          pl.BlockSpec((1, D), lambda i: (0, 0)),
          pl.BlockSpec((D, D), lambda i: (0, 0)),
      ],
      out_specs=[
          pl.BlockSpec((BR, D), lambda i: (i, 0)),
          pl.BlockSpec((BR, 8), lambda i: (i, 0)),
      ],
      out_shape=[
          jax.ShapeDtypeStruct((N_PAD, D), _f32),
          jax.ShapeDtypeStruct((N_PAD, 8), _f32),
      ],
      compiler_params=pltpu.CompilerParams(
          dimension_semantics=("arbitrary",)),
  )(x_pad, deg2, w_emb, b_emb, w0)


def _tc_layer_body(acc_ref, zp_ref, dinv_ref, b_ref, g_ref, be_ref, w_ref,
                   out_ref, stats_ref, pre_ref):
  p = pl.program_id(0)
  i = pl.program_id(1)
  dinv_col = dinv_ref[...][:, :1]

  @pl.when(p == 0)
  def _():
    pre = dinv_col * (acc_ref[0] + acc_ref[1] + zp_ref[...]) + b_ref[...]
    pre_ref[pl.ds(i, 1)] = pre.reshape(1, BR, D)

    @pl.when(i == 0)
    def _():
      stats_ref[...] = jnp.zeros((8, 128), _f32)
    pm = jnp.where(_row_mask(i), pre, 0.0)
    stats_ref[pl.ds(0, 1), :] += jnp.sum(pm, axis=0, keepdims=True)
    stats_ref[pl.ds(1, 1), :] += jnp.sum(pm * pm, axis=0, keepdims=True)

  @pl.when(p == 1)
  def _():
    inv_n = 1.0 / N
    mu = stats_ref[pl.ds(0, 1), :] * inv_n
    var = stats_ref[pl.ds(1, 1), :] * inv_n - mu * mu
    scale = g_ref[...] * lax.rsqrt(var + 1e-5)
    pre = pre_ref[pl.ds(i, 1)].reshape(BR, D)
    h = jnp.maximum((pre - mu) * scale + be_ref[...], 0.0)
    z = _DOT(h, w_ref[...])
    out_ref[...] = jnp.where(_row_mask(i), dinv_col * z, 0.0)


def _tc_layer(acc, zp, dinv, b, g, be, w):
  return pl.pallas_call(
      _tc_layer_body,
      grid=(2, GB),
      in_specs=[
          pl.BlockSpec((NC, BR, D), lambda p, i: (0, i * (1 - p), 0)),
          pl.BlockSpec((BR, D), lambda p, i: (i * (1 - p), 0)),
          pl.BlockSpec((BR, 8), lambda p, i: (i, 0)),
          pl.BlockSpec((1, D), lambda p, i: (0, 0)),
          pl.BlockSpec((1, D), lambda p, i: (0, 0)),
          pl.BlockSpec((1, D), lambda p, i: (0, 0)),
          pl.BlockSpec((D, D), lambda p, i: (0, 0)),
      ],
      out_specs=pl.BlockSpec((BR, D), lambda p, i: (i, 0)),
      out_shape=jax.ShapeDtypeStruct((N_PAD, D), _f32),
      scratch_shapes=[pltpu.VMEM((8, 128), _f32),
                      pltpu.VMEM((GB, BR, D), _f32)],
      compiler_params=pltpu.CompilerParams(
          dimension_semantics=("arbitrary", "arbitrary")),
  )(acc, zp, dinv, b, g, be, w)


def _tc_pool_body(acc_ref, zp_ref, dinv_ref, b_ref, batch_ref, out_ref,
                  cnt_ref):
  i = pl.program_id(0)
  dinv_col = dinv_ref[...][:, :1]
  ssum = acc_ref[0] + acc_ref[1] + zp_ref[...]
  pre = dinv_col * ssum + b_ref[...]

  brow = batch_ref[0]
  gid = lax.broadcasted_iota(jnp.int32, (G, BR), 0)
  onehot = (gid == jnp.broadcast_to(brow, (G, BR))).astype(_f32)

  @pl.when(i == 0)
  def _():
    out_ref[...] = jnp.zeros((G, D), _f32)
    cnt_ref[...] = jnp.zeros((G, 128), _f32)

  out_ref[...] += _DOT(onehot, pre)
  cnt_ref[...] += jnp.broadcast_to(
      jnp.sum(onehot, axis=1, keepdims=True), (G, 128))

  @pl.when(i == GB - 1)
  def _():
    out_ref[...] = out_ref[...] / jnp.maximum(cnt_ref[...], 1.0)


def _tc_pool(acc, zp, dinv, b, batch3):
  return pl.pallas_call(
      _tc_pool_body,
      grid=(GB,),
      in_specs=[
          pl.BlockSpec((NC, BR, D), lambda i: (0, i, 0)),
          pl.BlockSpec((BR, D), lambda i: (i, 0)),
          pl.BlockSpec((BR, 8), lambda i: (i, 0)),
          pl.BlockSpec((1, D), lambda i: (0, 0)),
          pl.BlockSpec((1, 1, BR), lambda i: (i, 0, 0)),
      ],
      out_specs=pl.BlockSpec((G, D), lambda i: (0, 0)),
      out_shape=jax.ShapeDtypeStruct((G, D), _f32),
      scratch_shapes=[pltpu.VMEM((G, 128), _f32)],
      compiler_params=pltpu.CompilerParams(
          dimension_semantics=("arbitrary",)),
  )(acc, zp, dinv, b, batch3)


def kernel(x, edge_index, batch, W_emb, b_emb, W0, b0, g0, be0,
           W1, b1, g1, be1, W2, b2):
  n_extra = E_PAD - E
  pad_src = jnp.arange(n_extra, dtype=jnp.int32) % N
  pad_dst = N + 1 + jnp.arange(n_extra, dtype=jnp.int32) % (N_PAD - N - 1)
  src3 = jnp.concatenate([edge_index[0], pad_src]).reshape(NW, STEPS, KW)
  dst3 = jnp.concatenate([edge_index[1], pad_dst]).reshape(NW, STEPS, KW)

  x_pad = jnp.concatenate([x.astype(_f32),
                           jnp.zeros((N_PAD - N, D), _f32)], axis=0)
  batch3 = jnp.concatenate(
      [batch.astype(jnp.int32),
       jnp.full((N_PAD - N,), 127, jnp.int32)]).reshape(GB, 1, BR)

  r1 = lambda v: v.astype(_f32).reshape(1, D)

  deg2 = _sc_degree(dst3).reshape(NC, RB, 128)
  zp0, dinv = _tc_embed(x_pad, deg2, W_emb.astype(_f32), r1(b_emb),
                        W0.astype(_f32))

  acc = _sc_aggregate(zp0, src3, dst3)
  zp1 = _tc_layer(acc, zp0, dinv, r1(b0), r1(g0), r1(be0), W1.astype(_f32))

  acc = _sc_aggregate(zp1, src3, dst3)
  zp2 = _tc_layer(acc, zp1, dinv, r1(b1), r1(g1), r1(be1), W2.astype(_f32))

  acc = _sc_aggregate(zp2, src3, dst3)
  return _tc_pool(acc, zp2, dinv, r1(b2), batch3)


# speedup vs baseline: 1.5142x; 1.0495x over previous
import functools

import jax
import jax.numpy as jnp
from jax import lax
from jax.experimental import pallas as pl
from jax.experimental.pallas import tpu as pltpu
from jax.experimental.pallas import tpu_sc as plsc

N = 10000
E = 320000
D = 128
G = 64

NC = 2
NS = 16
NW = NC * NS

KW = 128
STEPS = 80
E_PAD = NW * STEPS * KW
N_PAD = 10240
RB = N_PAD // 128
RPS = N_PAD // NS

_f32 = jnp.float32


def _sc_mesh():
  return plsc.VectorSubcoreMesh(core_axis_name="c", subcore_axis_name="s",
                                num_cores=NC, num_subcores=NS)


def _sc_degree_body(dst_hbm, deg_out, idx_v, ones_v, zero_v, deg_sh, sem):
  c = lax.axis_index("c")
  s = lax.axis_index("s")
  wid = s * NC + c

  for j in range(8):
    ones_v[pl.ds(16 * j, 16)] = jnp.ones((16,), _f32)
  for j in range(RPS // 16):
    zero_v[pl.ds(16 * j, 16)] = jnp.zeros((16,), _f32)
  pltpu.sync_copy(zero_v, deg_sh.at[pl.ds(s * RPS, RPS)])
  plsc.subcore_barrier()

  pltpu.sync_copy(dst_hbm.at[wid], idx_v)
  for g in range(STEPS // 8):
    descs = []
    for j in range(8):
      descs.append(
          pltpu.async_copy(ones_v, deg_sh.at[idx_v.at[8 * g + j]], sem,
                           add=True))
    for d in descs:
      d.wait()
  plsc.subcore_barrier()

  pltpu.sync_copy(deg_sh.at[pl.ds(s * RPS, RPS)],
                  deg_out.at[c, pl.ds(s * RPS, RPS)])


def _sc_degree(dst3):
  return pl.kernel(
      _sc_degree_body,
      out_type=jax.ShapeDtypeStruct((NC, N_PAD), _f32),
      mesh=_sc_mesh(),
      scratch_types=[
          pltpu.VMEM((STEPS, KW), jnp.int32),
          pltpu.VMEM((KW,), _f32),
          pltpu.VMEM((RPS,), _f32),
          pltpu.VMEM_SHARED((N_PAD,), _f32),
          pltpu.SemaphoreType.DMA,
      ],
  )(dst3)


PHASE = 40


def _sc_aggregate_body(zp_hbm, src_hbm, dst_hbm, acc_out,
                       src_idx, dst_idx, rows, acc_sh, gsems, ssems):
  c = lax.axis_index("c")
  s = lax.axis_index("s")
  wid = s * NC + c

  def _zero_row(r, carry):
    for j in range(8):
      rows[0, r, pl.ds(16 * j, 16)] = jnp.zeros((16,), _f32)
    return carry
  lax.fori_loop(0, 128, _zero_row, 0)
  for k in range(RPS // 128):
    pltpu.sync_copy(rows.at[0], acc_sh.at[pl.ds(s * RPS + 128 * k, 128)])
  plsc.subcore_barrier()

  gd = [None, None]
  sd = [None, None]
  for phase in range(STEPS // PHASE):
    pltpu.sync_copy(src_hbm.at[wid, pl.ds(PHASE * phase, PHASE)], src_idx)
    pltpu.sync_copy(dst_hbm.at[wid, pl.ds(PHASE * phase, PHASE)], dst_idx)
    for b in range(2):
      gd[b] = pltpu.async_copy(zp_hbm.at[src_idx.at[b]], rows.at[b],
                               gsems[b])
    for it in range(PHASE + 1):
      if it < PHASE:
        b = it % 2
        gd[b].wait()
        sd[b] = pltpu.async_copy(rows.at[b], acc_sh.at[dst_idx.at[it]],
                                 ssems[b], add=True)
      k = it + 1
      if 2 <= k < PHASE:
        bb = k % 2
        sd[bb].wait()
        gd[bb] = pltpu.async_copy(zp_hbm.at[src_idx.at[k]], rows.at[bb],
                                  gsems[bb])
    sd[(PHASE - 2) % 2].wait()
    sd[(PHASE - 1) % 2].wait()
  plsc.subcore_barrier()

  for k in range(RPS // 128):
    pltpu.sync_copy(acc_sh.at[pl.ds(s * RPS + 128 * k, 128)],
                    acc_out.at[c, pl.ds(s * RPS + 128 * k, 128)])


def _sc_aggregate(zp, src3, dst3):
  body = lambda zp_hbm, src_hbm, dst_hbm, acc_out, src_idx, dst_idx, rows, \
      acc_sh, g0, g1, s0, s1: _sc_aggregate_body(
          zp_hbm, src_hbm, dst_hbm, acc_out, src_idx, dst_idx, rows,
          acc_sh, [g0, g1], [s0, s1])
  return pl.kernel(
      body,
      out_type=jax.ShapeDtypeStruct((NC, N_PAD, D), _f32),
      mesh=_sc_mesh(),
      scratch_types=[
          pltpu.VMEM((PHASE, KW), jnp.int32),
          pltpu.VMEM((PHASE, KW), jnp.int32),
          pltpu.VMEM((2, KW, D), _f32),
          pltpu.VMEM_SHARED((N_PAD, D), _f32),
      ] + [pltpu.SemaphoreType.DMA] * 4,
  )(zp, src3, dst3)


_DOT = functools.partial(jnp.dot, preferred_element_type=_f32)

BR = 1280
GB = N_PAD // BR
SUB = BR // 128


def _row_mask(i):
  return (i * BR + lax.broadcasted_iota(jnp.int32, (BR, 128), 0)) < N


def _tc_embed_body(x_ref, deg_ref, wemb_ref, bemb_ref, w0_ref,
                   zp_ref, dinv_ref):
  i = pl.program_id(0)
  degsum = (deg_ref[0, pl.ds(SUB * i, SUB), :]
            + deg_ref[1, pl.ds(SUB * i, SUB), :] + 1.0)
  dinv = lax.rsqrt(degsum)
  row = lax.broadcasted_iota(jnp.int32, (128, 128), 0)
  col = lax.broadcasted_iota(jnp.int32, (128, 128), 1)
  cols = []
  for k in range(SUB):
    dk = dinv[pl.ds(k, 1) if False else k]
    diag = jnp.where(row == col,
                     jnp.broadcast_to(dk.reshape(1, 128), (128, 128)), 0.0)
    cols.append(_DOT(diag, jnp.ones((128, 8), _f32)))
  dinv_col = jnp.concatenate(cols, axis=0)
  dinv_ref[...] = dinv_col
  h = _DOT(x_ref[...], wemb_ref[...]) + bemb_ref[...]
  z = _DOT(h, w0_ref[...])
  zp_ref[...] = jnp.where(_row_mask(i), dinv_col[:, :1] * z, 0.0)


def _tc_embed(x_pad, deg2, w_emb, b_emb, w0):
  return pl.pallas_call(
      _tc_embed_body,
      grid=(GB,),
      in_specs=[
          pl.BlockSpec((BR, D), lambda i: (i, 0)),
          pl.BlockSpec((NC, RB, 128), lambda i: (0, 0, 0)),
          pl.BlockSpec((D, D), lambda i: (0, 0)),
          pl.BlockSpec((1, D), lambda i: (0, 0)),
          pl.BlockSpec((D, D), lambda i: (0, 0)),
      ],
      out_specs=[
          pl.BlockSpec((BR, D), lambda i: (i, 0)),
          pl.BlockSpec((BR, 8), lambda i: (i, 0)),
      ],
      out_shape=[
          jax.ShapeDtypeStruct((N_PAD, D), _f32),
          jax.ShapeDtypeStruct((N_PAD, 8), _f32),
      ],
      compiler_params=pltpu.CompilerParams(
          dimension_semantics=("arbitrary",)),
  )(x_pad, deg2, w_emb, b_emb, w0)


def _tc_layer_body(acc_ref, zp_ref, dinv_ref, b_ref, g_ref, be_ref, w_ref,
                   out_ref, stats_ref, pre_ref):
  p = pl.program_id(0)
  i = pl.program_id(1)
  dinv_col = dinv_ref[...][:, :1]

  @pl.when(p == 0)
  def _():
    pre = dinv_col * (acc_ref[0] + acc_ref[1] + zp_ref[...]) + b_ref[...]
    pre_ref[pl.ds(i, 1)] = pre.reshape(1, BR, D)

    @pl.when(i == 0)
    def _():
      stats_ref[...] = jnp.zeros((8, 128), _f32)
    pm = jnp.where(_row_mask(i), pre, 0.0)
    stats_ref[pl.ds(0, 1), :] += jnp.sum(pm, axis=0, keepdims=True)
    stats_ref[pl.ds(1, 1), :] += jnp.sum(pm * pm, axis=0, keepdims=True)

  @pl.when(p == 1)
  def _():
    inv_n = 1.0 / N
    mu = stats_ref[pl.ds(0, 1), :] * inv_n
    var = stats_ref[pl.ds(1, 1), :] * inv_n - mu * mu
    scale = g_ref[...] * lax.rsqrt(var + 1e-5)
    pre = pre_ref[pl.ds(i, 1)].reshape(BR, D)
    h = jnp.maximum((pre - mu) * scale + be_ref[...], 0.0)
    z = _DOT(h, w_ref[...])
    out_ref[...] = jnp.where(_row_mask(i), dinv_col * z, 0.0)


def _tc_layer(acc, zp, dinv, b, g, be, w):
  return pl.pallas_call(
      _tc_layer_body,
      grid=(2, GB),
      in_specs=[
          pl.BlockSpec((NC, BR, D), lambda p, i: (0, i * (1 - p), 0)),
          pl.BlockSpec((BR, D), lambda p, i: (i * (1 - p), 0)),
          pl.BlockSpec((BR, 8), lambda p, i: (i, 0)),
          pl.BlockSpec((1, D), lambda p, i: (0, 0)),
          pl.BlockSpec((1, D), lambda p, i: (0, 0)),
          pl.BlockSpec((1, D), lambda p, i: (0, 0)),
          pl.BlockSpec((D, D), lambda p, i: (0, 0)),
      ],
      out_specs=pl.BlockSpec((BR, D), lambda p, i: (i, 0)),
      out_shape=jax.ShapeDtypeStruct((N_PAD, D), _f32),
      scratch_shapes=[pltpu.VMEM((8, 128), _f32),
                      pltpu.VMEM((GB, BR, D), _f32)],
      compiler_params=pltpu.CompilerParams(
          dimension_semantics=("arbitrary", "arbitrary")),
  )(acc, zp, dinv, b, g, be, w)


def _tc_pool_body(acc_ref, zp_ref, dinv_ref, b_ref, batch_ref, out_ref,
                  cnt_ref):
  i = pl.program_id(0)
  dinv_col = dinv_ref[...][:, :1]
  ssum = acc_ref[0] + acc_ref[1] + zp_ref[...]
  pre = dinv_col * ssum + b_ref[...]

  brow = batch_ref[0]
  gid = lax.broadcasted_iota(jnp.int32, (G, BR), 0)
  onehot = (gid == jnp.broadcast_to(brow, (G, BR))).astype(_f32)

  @pl.when(i == 0)
  def _():
    out_ref[...] = jnp.zeros((G, D), _f32)
    cnt_ref[...] = jnp.zeros((G, 128), _f32)

  out_ref[...] += _DOT(onehot, pre)
  cnt_ref[...] += jnp.broadcast_to(
      jnp.sum(onehot, axis=1, keepdims=True), (G, 128))

  @pl.when(i == GB - 1)
  def _():
    out_ref[...] = out_ref[...] / jnp.maximum(cnt_ref[...], 1.0)


def _tc_pool(acc, zp, dinv, b, batch3):
  return pl.pallas_call(
      _tc_pool_body,
      grid=(GB,),
      in_specs=[
          pl.BlockSpec((NC, BR, D), lambda i: (0, i, 0)),
          pl.BlockSpec((BR, D), lambda i: (i, 0)),
          pl.BlockSpec((BR, 8), lambda i: (i, 0)),
          pl.BlockSpec((1, D), lambda i: (0, 0)),
          pl.BlockSpec((1, 1, BR), lambda i: (i, 0, 0)),
      ],
      out_specs=pl.BlockSpec((G, D), lambda i: (0, 0)),
      out_shape=jax.ShapeDtypeStruct((G, D), _f32),
      scratch_shapes=[pltpu.VMEM((G, 128), _f32)],
      compiler_params=pltpu.CompilerParams(
          dimension_semantics=("arbitrary",)),
  )(acc, zp, dinv, b, batch3)


def kernel(x, edge_index, batch, W_emb, b_emb, W0, b0, g0, be0,
           W1, b1, g1, be1, W2, b2):
  n_extra = E_PAD - E
  pad_src = jnp.arange(n_extra, dtype=jnp.int32) % N
  pad_dst = N + 1 + jnp.arange(n_extra, dtype=jnp.int32) % (N_PAD - N - 1)
  src3 = jnp.concatenate([edge_index[0], pad_src]).reshape(NW, STEPS, KW)
  dst3 = jnp.concatenate([edge_index[1], pad_dst]).reshape(NW, STEPS, KW)

  x_pad = jnp.concatenate([x.astype(_f32),
                           jnp.zeros((N_PAD - N, D), _f32)], axis=0)
  batch3 = jnp.concatenate(
      [batch.astype(jnp.int32),
       jnp.full((N_PAD - N,), 127, jnp.int32)]).reshape(GB, 1, BR)

  r1 = lambda v: v.astype(_f32).reshape(1, D)

  deg2 = _sc_degree(dst3).reshape(NC, RB, 128)
  zp0, dinv = _tc_embed(x_pad, deg2, W_emb.astype(_f32), r1(b_emb),
                        W0.astype(_f32))

  acc = _sc_aggregate(zp0, src3, dst3)
  zp1 = _tc_layer(acc, zp0, dinv, r1(b0), r1(g0), r1(be0), W1.astype(_f32))

  acc = _sc_aggregate(zp1, src3, dst3)
  zp2 = _tc_layer(acc, zp1, dinv, r1(b1), r1(g1), r1(be1), W2.astype(_f32))

  acc = _sc_aggregate(zp2, src3, dst3)
  return _tc_pool(acc, zp2, dinv, r1(b2), batch3)


# speedup vs baseline: 1.5388x; 1.0163x over previous
import functools

import jax
import jax.numpy as jnp
from jax import lax
from jax.experimental import pallas as pl
from jax.experimental.pallas import tpu as pltpu
from jax.experimental.pallas import tpu_sc as plsc

N = 10000
E = 320000
D = 128
G = 64

NC = 2
NS = 16
NW = NC * NS

KW = 128
STEPS = 80
E_PAD = NW * STEPS * KW
N_PAD = 10240
RB = N_PAD // 128
RPS = N_PAD // NS

_f32 = jnp.float32


def _sc_mesh():
  return plsc.VectorSubcoreMesh(core_axis_name="c", subcore_axis_name="s",
                                num_cores=NC, num_subcores=NS)


def _sc_degree_body(dst_hbm, deg_out, idx_v, ones_v, zero_v, deg_sh, sem):
  c = lax.axis_index("c")
  s = lax.axis_index("s")
  wid = s * NC + c

  for j in range(8):
    ones_v[pl.ds(16 * j, 16)] = jnp.ones((16,), _f32)
  for j in range(RPS // 16):
    zero_v[pl.ds(16 * j, 16)] = jnp.zeros((16,), _f32)
  pltpu.sync_copy(zero_v, deg_sh.at[pl.ds(s * RPS, RPS)])
  plsc.subcore_barrier()

  pltpu.sync_copy(dst_hbm.at[wid], idx_v)
  for g in range(STEPS // 8):
    descs = []
    for j in range(8):
      descs.append(
          pltpu.async_copy(ones_v, deg_sh.at[idx_v.at[8 * g + j]], sem,
                           add=True))
    for d in descs:
      d.wait()
  plsc.subcore_barrier()

  pltpu.sync_copy(deg_sh.at[pl.ds(s * RPS, RPS)],
                  deg_out.at[c, pl.ds(s * RPS, RPS)])


def _sc_degree(dst3):
  return pl.kernel(
      _sc_degree_body,
      out_type=jax.ShapeDtypeStruct((NC, N_PAD), _f32),
      mesh=_sc_mesh(),
      scratch_types=[
          pltpu.VMEM((STEPS, KW), jnp.int32),
          pltpu.VMEM((KW,), _f32),
          pltpu.VMEM((RPS,), _f32),
          pltpu.VMEM_SHARED((N_PAD,), _f32),
          pltpu.SemaphoreType.DMA,
      ],
  )(dst3)


PHASE = 40


def _sc_aggregate_body(zp_hbm, src_hbm, dst_hbm, acc_out,
                       src_idx, dst_idx, rows, acc_sh, gsems, ssems):
  c = lax.axis_index("c")
  s = lax.axis_index("s")
  wid = s * NC + c

  def _zero_row(r, carry):
    for j in range(8):
      rows[0, r, pl.ds(16 * j, 16)] = jnp.zeros((16,), _f32)
    return carry
  lax.fori_loop(0, 128, _zero_row, 0)
  for k in range(RPS // 128):
    pltpu.sync_copy(rows.at[0], acc_sh.at[pl.ds(s * RPS + 128 * k, 128)])
  plsc.subcore_barrier()

  gd = [None, None]
  sd = [None, None]
  for phase in range(STEPS // PHASE):
    pltpu.sync_copy(src_hbm.at[wid, pl.ds(PHASE * phase, PHASE)], src_idx)
    pltpu.sync_copy(dst_hbm.at[wid, pl.ds(PHASE * phase, PHASE)], dst_idx)
    for b in range(2):
      gd[b] = pltpu.async_copy(zp_hbm.at[src_idx.at[b]], rows.at[b],
                               gsems[b])
    for it in range(PHASE + 1):
      if it < PHASE:
        b = it % 2
        gd[b].wait()
        sd[b] = pltpu.async_copy(rows.at[b], acc_sh.at[dst_idx.at[it]],
                                 ssems[b], add=True)
      k = it + 1
      if 2 <= k < PHASE:
        bb = k % 2
        sd[bb].wait()
        gd[bb] = pltpu.async_copy(zp_hbm.at[src_idx.at[k]], rows.at[bb],
                                  gsems[bb])
    sd[(PHASE - 2) % 2].wait()
    sd[(PHASE - 1) % 2].wait()
  plsc.subcore_barrier()

  for k in range(RPS // 128):
    pltpu.sync_copy(acc_sh.at[pl.ds(s * RPS + 128 * k, 128)],
                    acc_out.at[c, pl.ds(s * RPS + 128 * k, 128)])


def _sc_aggregate(zp, src3, dst3):
  body = lambda zp_hbm, src_hbm, dst_hbm, acc_out, src_idx, dst_idx, rows, \
      acc_sh, g0, g1, s0, s1: _sc_aggregate_body(
          zp_hbm, src_hbm, dst_hbm, acc_out, src_idx, dst_idx, rows,
          acc_sh, [g0, g1], [s0, s1])
  return pl.kernel(
      body,
      out_type=jax.ShapeDtypeStruct((NC, N_PAD, D), _f32),
      mesh=_sc_mesh(),
      scratch_types=[
          pltpu.VMEM((PHASE, KW), jnp.int32),
          pltpu.VMEM((PHASE, KW), jnp.int32),
          pltpu.VMEM((2, KW, D), _f32),
          pltpu.VMEM_SHARED((N_PAD, D), _f32),
      ] + [pltpu.SemaphoreType.DMA] * 4,
  )(zp, src3, dst3)


_DOT = functools.partial(jnp.dot, preferred_element_type=_f32)

BR = 1280
GB = N_PAD // BR
SUB = BR // 128


def _row_mask(i):
  return (i * BR + lax.broadcasted_iota(jnp.int32, (BR, 128), 0)) < N


def _tc_embed_body(x_ref, deg_ref, wemb_ref, bemb_ref, w0_ref,
                   zp_ref, dinv_ref):
  i = pl.program_id(0)
  degsum = (deg_ref[0, pl.ds(SUB * i, SUB), :]
            + deg_ref[1, pl.ds(SUB * i, SUB), :] + 1.0)
  dinv = lax.rsqrt(degsum)
  row = lax.broadcasted_iota(jnp.int32, (128, 128), 0)
  col = lax.broadcasted_iota(jnp.int32, (128, 128), 1)
  cols = []
  for k in range(SUB):
    dk = dinv[pl.ds(k, 1) if False else k]
    diag = jnp.where(row == col,
                     jnp.broadcast_to(dk.reshape(1, 128), (128, 128)), 0.0)
    cols.append(_DOT(diag, jnp.ones((128, 8), _f32)))
  dinv_col = jnp.concatenate(cols, axis=0)
  dinv_ref[...] = dinv_col
  h = _DOT(x_ref[...], wemb_ref[...]) + bemb_ref[...]
  z = _DOT(h, w0_ref[...])
  zp_ref[...] = jnp.where(_row_mask(i), dinv_col[:, :1] * z, 0.0)


def _tc_embed(x_pad, deg2, w_emb, b_emb, w0):
  return pl.pallas_call(
      _tc_embed_body,
      grid=(GB,),
      in_specs=[
          pl.BlockSpec((BR, D), lambda i: (i, 0)),
          pl.BlockSpec((NC, RB, 128), lambda i: (0, 0, 0)),
          pl.BlockSpec((D, D), lambda i: (0, 0)),
          pl.BlockSpec((1, D), lambda i: (0, 0)),
          pl.BlockSpec((D, D), lambda i: (0, 0)),
      ],
      out_specs=[
          pl.BlockSpec((BR, D), lambda i: (i, 0)),
          pl.BlockSpec((BR, 8), lambda i: (i, 0)),
      ],
      out_shape=[
          jax.ShapeDtypeStruct((N_PAD, D), _f32),
          jax.ShapeDtypeStruct((N_PAD, 8), _f32),
      ],
      compiler_params=pltpu.CompilerParams(
          dimension_semantics=("arbitrary",)),
  )(x_pad, deg2, w_emb, b_emb, w0)


def _tc_layer_body(acc_ref, zp_ref, dinv_ref, b_ref, g_ref, be_ref, w_ref,
                   out_ref, stats_ref, pre_ref):
  p = pl.program_id(0)
  i = pl.program_id(1)
  dinv_col = dinv_ref[...][:, :1]

  @pl.when(p == 0)
  def _():
    pre = dinv_col * (acc_ref[0] + acc_ref[1] + zp_ref[...]) + b_ref[...]
    pre_ref[pl.ds(i, 1)] = pre.reshape(1, BR, D)

    @pl.when(i == 0)
    def _():
      stats_ref[...] = jnp.zeros((8, 128), _f32)
    pm = jnp.where(_row_mask(i), pre, 0.0)
    stats_ref[pl.ds(0, 1), :] += jnp.sum(pm, axis=0, keepdims=True)
    stats_ref[pl.ds(1, 1), :] += jnp.sum(pm * pm, axis=0, keepdims=True)

  @pl.when(p == 1)
  def _():
    inv_n = 1.0 / N
    mu = stats_ref[pl.ds(0, 1), :] * inv_n
    var = stats_ref[pl.ds(1, 1), :] * inv_n - mu * mu
    scale = g_ref[...] * lax.rsqrt(var + 1e-5)
    pre = pre_ref[pl.ds(i, 1)].reshape(BR, D)
    h = jnp.maximum((pre - mu) * scale + be_ref[...], 0.0)
    z = _DOT(h, w_ref[...])
    out_ref[...] = jnp.where(_row_mask(i), dinv_col * z, 0.0)


def _tc_layer(acc, zp, dinv, b, g, be, w):
  return pl.pallas_call(
      _tc_layer_body,
      grid=(2, GB),
      in_specs=[
          pl.BlockSpec((NC, BR, D), lambda p, i: (0, i * (1 - p), 0)),
          pl.BlockSpec((BR, D), lambda p, i: (i * (1 - p), 0)),
          pl.BlockSpec((BR, 8), lambda p, i: (i, 0)),
          pl.BlockSpec((1, D), lambda p, i: (0, 0)),
          pl.BlockSpec((1, D), lambda p, i: (0, 0)),
          pl.BlockSpec((1, D), lambda p, i: (0, 0)),
          pl.BlockSpec((D, D), lambda p, i: (0, 0)),
      ],
      out_specs=pl.BlockSpec((BR, D), lambda p, i: (i, 0)),
      out_shape=jax.ShapeDtypeStruct((N_PAD, D), _f32),
      scratch_shapes=[pltpu.VMEM((8, 128), _f32),
                      pltpu.VMEM((GB, BR, D), _f32)],
      compiler_params=pltpu.CompilerParams(
          dimension_semantics=("arbitrary", "arbitrary")),
  )(acc, zp, dinv, b, g, be, w)


def _tc_pool_body(acc_ref, zp_ref, dinv_ref, b_ref, batch_ref, out_ref,
                  cnt_ref):
  i = pl.program_id(0)
  dinv_col = dinv_ref[...][:, :1]
  ssum = acc_ref[0] + acc_ref[1] + zp_ref[...]
  pre = dinv_col * ssum + b_ref[...]

  brow = batch_ref[0]
  gid = lax.broadcasted_iota(jnp.int32, (G, BR), 0)
  onehot = (gid == jnp.broadcast_to(brow, (G, BR))).astype(_f32)

  @pl.when(i == 0)
  def _():
    out_ref[...] = jnp.zeros((G, D), _f32)
    cnt_ref[...] = jnp.zeros((G, 128), _f32)

  out_ref[...] += _DOT(onehot, pre)
  cnt_ref[...] += jnp.broadcast_to(
      jnp.sum(onehot, axis=1, keepdims=True), (G, 128))

  @pl.when(i == GB - 1)
  def _():
    out_ref[...] = out_ref[...] / jnp.maximum(cnt_ref[...], 1.0)


def _tc_pool(acc, zp, dinv, b, batch3):
  return pl.pallas_call(
      _tc_pool_body,
      grid=(GB,),
      in_specs=[
          pl.BlockSpec((NC, BR, D), lambda i: (0, i, 0)),
          pl.BlockSpec((BR, D), lambda i: (i, 0)),
          pl.BlockSpec((BR, 8), lambda i: (i, 0)),
          pl.BlockSpec((1, D), lambda i: (0, 0)),
          pl.BlockSpec((1, 1, BR), lambda i: (i, 0, 0)),
      ],
      out_specs=pl.BlockSpec((G, D), lambda i: (0, 0)),
      out_shape=jax.ShapeDtypeStruct((G, D), _f32),
      scratch_shapes=[pltpu.VMEM((G, 128), _f32)],
      compiler_params=pltpu.CompilerParams(
          dimension_semantics=("arbitrary",)),
  )(acc, zp, dinv, b, batch3)


def kernel(x, edge_index, batch, W_emb, b_emb, W0, b0, g0, be0,
           W1, b1, g1, be1, W2, b2):
  n_extra = E_PAD - E
  pad_src = jnp.arange(n_extra, dtype=jnp.int32) % N
  pad_dst = N + 1 + jnp.arange(n_extra, dtype=jnp.int32) % (N_PAD - N - 1)
  edges = jnp.concatenate(
      [edge_index.astype(jnp.int32),
       jnp.stack([pad_src, pad_dst])], axis=1).reshape(2, NW, STEPS, KW)
  src3 = edges[0]
  dst3 = edges[1]
  batch3 = jnp.concatenate(
      [batch.astype(jnp.int32),
       jnp.full((N_PAD - N,), 127, jnp.int32)]).reshape(GB, 1, BR)

  r1 = lambda v: v.astype(_f32).reshape(1, D)

  deg2 = _sc_degree(dst3).reshape(NC, RB, 128)
  zp0, dinv = _tc_embed(x.astype(_f32), deg2, W_emb.astype(_f32), r1(b_emb),
                        W0.astype(_f32))

  acc = _sc_aggregate(zp0, src3, dst3)
  zp1 = _tc_layer(acc, zp0, dinv, r1(b0), r1(g0), r1(be0), W1.astype(_f32))

  acc = _sc_aggregate(zp1, src3, dst3)
  zp2 = _tc_layer(acc, zp1, dinv, r1(b1), r1(g1), r1(be1), W2.astype(_f32))

  acc = _sc_aggregate(zp2, src3, dst3)
  return _tc_pool(acc, zp2, dinv, r1(b2), batch3)
